# baseline (speedup 1.0000x reference)
import functools

import jax
import jax.numpy as jnp
from jax import lax
from jax.experimental import pallas as pl
from jax.experimental.pallas import tpu as pltpu
from jax.experimental.pallas import tpu_sc as plsc

N = 10000
E = 320000
H = 128
C = 16
EPS = 0.5

NP = 10112
NW = 32
B = 128
NB = 80
PER_W = NB * B
EP = NW * PER_W
RPT = NP // 16

_mesh = plsc.VectorSubcoreMesh(
    core_axis_name="c", subcore_axis_name="s", num_cores=2, num_subcores=16)

_f32 = jnp.float32


def _hist_body(rc_hbm, deg_out, cnt_out, dacc, cacc, idxb, ones_v):
    cid = lax.axis_index("c")
    sid = lax.axis_index("s")
    wid = cid * 16 + sid
    zero_row = jnp.zeros((16,), _f32)
    one_row = jnp.ones((16,), _f32)

    for g in range(B // 16):
        ones_v[pl.ds(g * 16, 16)] = zero_row
    r0 = sid * RPT
    for k in range(4):
        pltpu.sync_copy(ones_v.at[pl.ds(0, B)], dacc.at[pl.ds(r0 + k * B, B)])
        pltpu.sync_copy(ones_v.at[pl.ds(0, B)], cacc.at[pl.ds(r0 + k * B, B)])
    pltpu.sync_copy(ones_v.at[pl.ds(0, RPT - 4 * B)],
                    dacc.at[pl.ds(r0 + 4 * B, RPT - 4 * B)])
    pltpu.sync_copy(ones_v.at[pl.ds(0, RPT - 4 * B)],
                    cacc.at[pl.ds(r0 + 4 * B, RPT - 4 * B)])
    for g in range(B // 16):
        ones_v[pl.ds(g * 16, 16)] = one_row
    plsc.subcore_barrier()

    def _blk(k, carry):
        pltpu.sync_copy(rc_hbm.at[wid * NB + k], idxb)
        pltpu.sync_copy(ones_v.at[pl.ds(0, B)], dacc.at[idxb.at[0]],
                        add=True)
        pltpu.sync_copy(ones_v.at[pl.ds(0, B)], cacc.at[idxb.at[1]],
                        add=True)
        return carry
    lax.fori_loop(0, NB, _blk, 0)
    plsc.subcore_barrier()

    pltpu.sync_copy(dacc.at[pl.ds(r0, RPT)], deg_out.at[cid, pl.ds(r0, RPT)])
    pltpu.sync_copy(cacc.at[pl.ds(r0, RPT)], cnt_out.at[cid, pl.ds(r0, RPT)])


_hist = functools.partial(
    pl.kernel,
    out_type=(jax.ShapeDtypeStruct((2, NP), _f32),
              jax.ShapeDtypeStruct((2, NP), _f32)),
    mesh=_mesh,
    compiler_params=pltpu.CompilerParams(
        needs_layout_passes=False, use_tc_tiling_on_sc=False),
    scratch_types=[
        pltpu.MemorySpace.VMEM_SHARED((NP,), _f32),
        pltpu.MemorySpace.VMEM_SHARED((NP,), _f32),
        pltpu.VMEM((2, B), jnp.int32),
        pltpu.VMEM((B,), _f32),
    ],
)(_hist_body)


def _edge_body(rc_hbm, table_hbm, a_hbm, b_hbm, d_hbm,
               part_out, s_out, accp, accs, a_sp, b_sp, d_sp,
               idxb, gbuf, arb, bcb, drb, dcb, coef_v, st_v,
               gsem0, gsem1):
    cid = lax.axis_index("c")
    sid = lax.axis_index("s")
    wid = cid * 16 + sid
    zero_row = jnp.zeros((16,), _f32)
    r0 = sid * RPT

    ch_h = pl.ds(r0, RPT)
    for (hbm, sp) in ((a_hbm, a_sp), (b_hbm, b_sp), (d_hbm, d_sp)):
        pltpu.sync_copy(hbm.at[ch_h], st_v)
        pltpu.sync_copy(st_v, sp.at[ch_h])

    def _zrow(i, carry):
        for j in range(8):
            gbuf[0, i, pl.ds(j * 16, 16)] = zero_row
        return carry
    lax.fori_loop(0, B, _zrow, 0)
    for g in range(B // 16):
        coef_v[0, pl.ds(g * 16, 16)] = zero_row
    for k in range(4):
        pltpu.sync_copy(gbuf.at[0], accp.at[pl.ds(r0 + k * B, B)])
        pltpu.sync_copy(coef_v.at[0], accs.at[pl.ds(r0 + k * B, B)])
    pltpu.sync_copy(gbuf.at[0, pl.ds(0, RPT - 4 * B)],
                    accp.at[pl.ds(r0 + 4 * B, RPT - 4 * B)])
    pltpu.sync_copy(coef_v.at[0, pl.ds(0, RPT - 4 * B)],
                    accs.at[pl.ds(r0 + 4 * B, RPT - 4 * B)])
    plsc.subcore_barrier()

    def _tg_desc(slot, p, gsem):
        return pltpu.make_async_copy(table_hbm.at[idxb.at[slot, 0]],
                                     gbuf.at[p], gsem)

    lane = lax.broadcasted_iota(jnp.int32, (16,), 0)
    z16 = lane * 0
    o16 = z16 + 1

    pltpu.sync_copy(rc_hbm.at[wid * NB], idxb.at[0])
    _tg_desc(0, 0, gsem0).start()

    def _kk(kk, carry):
        for ph in range(2):
            k = 2 * kk + ph
            p = ph
            q = 1 - ph
            gsem_p = gsem0 if p == 0 else gsem1
            gsem_q = gsem1 if p == 0 else gsem0

            kn = jnp.minimum(k + 1, NB - 1)
            pltpu.sync_copy(rc_hbm.at[wid * NB + kn], idxb.at[q])
            _tg_desc(q, q, gsem_q).start()

            pltpu.sync_copy(a_sp.at[idxb.at[p, 0]], arb)
            pltpu.sync_copy(d_sp.at[idxb.at[p, 0]], drb)
            pltpu.sync_copy(b_sp.at[idxb.at[p, 1]], bcb)
            pltpu.sync_copy(d_sp.at[idxb.at[p, 1]], dcb)

            for g in range(B // 16):
                sl = pl.ds(g * 16, 16)
                xv = arb[sl] + bcb[sl]
                ex = jnp.exp(-2.0 * jnp.abs(xv))
                thv = jnp.sign(xv) * (1.0 - ex) / (1.0 + ex)
                coef_v[p, sl] = thv * (drb[sl] * dcb[sl])

            _tg_desc(p, p, gsem_p).wait()

            def _scaleg(g, carry2):
                cvec = coef_v[p, pl.ds(g * 16, 16)]
                for e16 in range(16):
                    e = g * 16 + e16
                    cf = cvec[e16]
                    for j in range(8):
                        gbuf[p, e, pl.ds(j * 16, 16)] = (
                            gbuf[p, e, pl.ds(j * 16, 16)] * cf)
                return carry2
            lax.fori_loop(0, B // 16, _scaleg, 0)

            pltpu.sync_copy(gbuf.at[p], accp.at[idxb.at[p, 1]], add=True)
            pltpu.sync_copy(coef_v.at[p], accs.at[idxb.at[p, 1]], add=True)
        return carry
    lax.fori_loop(0, NB // 2, _kk, 0)
    _tg_desc(0, 0, gsem0).wait()
    plsc.subcore_barrier()

    pltpu.sync_copy(accp.at[pl.ds(r0, RPT)], part_out.at[cid, pl.ds(r0, RPT)])
    pltpu.sync_copy(accs.at[pl.ds(r0, RPT)], s_out.at[cid, pl.ds(r0, RPT)])


_edge = functools.partial(
    pl.kernel,
    out_type=(jax.ShapeDtypeStruct((2, NP, H), _f32),
              jax.ShapeDtypeStruct((2, NP), _f32)),
    mesh=_mesh,
    compiler_params=pltpu.CompilerParams(
        needs_layout_passes=False, use_tc_tiling_on_sc=False),
    scratch_types=[
        pltpu.MemorySpace.VMEM_SHARED((NP, H), _f32),
        pltpu.MemorySpace.VMEM_SHARED((NP,), _f32),
        pltpu.MemorySpace.VMEM_SHARED((NP,), _f32),
        pltpu.MemorySpace.VMEM_SHARED((NP,), _f32),
        pltpu.MemorySpace.VMEM_SHARED((NP,), _f32),
        pltpu.VMEM((2, 2, B), jnp.int32),
        pltpu.VMEM((2, B, H), _f32),
        pltpu.VMEM((B,), _f32),
        pltpu.VMEM((B,), _f32),
        pltpu.VMEM((B,), _f32),
        pltpu.VMEM((B,), _f32),
        pltpu.VMEM((2, B), _f32),
        pltpu.VMEM((RPT,), _f32),
        pltpu.SemaphoreType.DMA,
        pltpu.SemaphoreType.DMA,
    ],
)(_edge_body)


_GRID = 10
_BR = N // _GRID


def _mm0_body(h_ref, W1_ref, b1_ref, Wg_ref, bgv_ref, hh_ref, g_ref):
    hh = jnp.maximum(
        jnp.dot(h_ref[...], W1_ref[...], preferred_element_type=_f32)
        + b1_ref[...], 0.0)
    hh_ref[...] = hh
    g_ref[...] = jnp.dot(hh, Wg_ref[...], preferred_element_type=_f32) + bgv_ref[...]


def _tc_mm0(h, W1, b1r, Wgcat, bgv):
    return pl.pallas_call(
        _mm0_body,
        grid=(_GRID,),
        in_specs=[
            pl.BlockSpec((_BR, H), lambda i: (i, 0)),
            pl.BlockSpec((H, H), lambda i: (0, 0)),
            pl.BlockSpec((1, H), lambda i: (0, 0)),
            pl.BlockSpec((H, 2), lambda i: (0, 0)),
            pl.BlockSpec((1, 2), lambda i: (0, 0)),
        ],
        out_specs=[
            pl.BlockSpec((_BR, H), lambda i: (i, 0)),
            pl.BlockSpec((_BR, 2), lambda i: (i, 0)),
        ],
        out_shape=[
            jax.ShapeDtypeStruct((NP, H), _f32),
            jax.ShapeDtypeStruct((NP, 2), _f32),
        ],
    )(h, W1, b1r, Wgcat, bgv)


def _prep_body(d_ref, c_ref, dinv_ref, cnti_ref):
    deg = d_ref[0] + d_ref[1]
    cnt = c_ref[0] + c_ref[1]
    rowid = lax.broadcasted_iota(jnp.int32, (NP, 1), 0)
    dinv_ref[...] = jnp.where(
        rowid < N, lax.rsqrt(jnp.maximum(deg, 1.0)), 0.0)
    cnti_ref[...] = 1.0 / jnp.maximum(cnt, 1.0)


def _tc_prep(degs, cnts):
    return pl.pallas_call(
        _prep_body,
        out_shape=[
            jax.ShapeDtypeStruct((NP, 1), _f32),
            jax.ShapeDtypeStruct((NP, 1), _f32),
        ],
    )(degs, cnts)


def _comb_body(lin_ref, raw_ref, p_ref, s_ref, ci_ref, Wg_ref, bgv_ref,
               hh2_ref, g_ref):
    P = p_ref[0] + p_ref[1]
    S = s_ref[0] + s_ref[1]
    out = (S * lin_ref[...] + P) * ci_ref[...]
    hh2 = EPS * raw_ref[...] + out
    hh2_ref[...] = hh2
    g_ref[...] = jnp.dot(hh2, Wg_ref[...], preferred_element_type=_f32) + bgv_ref[...]


def _tc_comb(layer_in, raw, part, svec, cntinv, Wgcat, bgv):
    return pl.pallas_call(
        _comb_body,
        grid=(_GRID,),
        in_specs=[
            pl.BlockSpec((_BR, H), lambda i: (i, 0)),
            pl.BlockSpec((_BR, H), lambda i: (i, 0)),
            pl.BlockSpec((2, _BR, H), lambda i: (0, i, 0)),
            pl.BlockSpec((2, _BR, 1), lambda i: (0, i, 0)),
            pl.BlockSpec((_BR, 1), lambda i: (i, 0)),
            pl.BlockSpec((H, 2), lambda i: (0, 0)),
            pl.BlockSpec((1, 2), lambda i: (0, 0)),
        ],
        out_specs=[
            pl.BlockSpec((_BR, H), lambda i: (i, 0)),
            pl.BlockSpec((_BR, 2), lambda i: (i, 0)),
        ],
        out_shape=[
            jax.ShapeDtypeStruct((NP, H), _f32),
            jax.ShapeDtypeStruct((NP, 2), _f32),
        ],
    )(layer_in, raw, part, svec, cntinv, Wgcat, bgv)


def _final_body(lin_ref, raw_ref, p_ref, s_ref, ci_ref, W2_ref, b2_ref,
                out_ref):
    P = p_ref[0] + p_ref[1]
    S = s_ref[0] + s_ref[1]
    out = (S * lin_ref[...] + P) * ci_ref[...]
    hh3 = EPS * raw_ref[...] + out
    logits = jnp.dot(hh3, W2_ref[...], preferred_element_type=_f32) + b2_ref[...]
    m = jnp.max(logits, axis=1, keepdims=True)
    lse = m + jnp.log(jnp.sum(jnp.exp(logits - m), axis=1, keepdims=True))
    out_ref[...] = logits - lse


def _tc_final(layer_in, raw, part, svec, cntinv, W2, b2r):
    return pl.pallas_call(
        _final_body,
        grid=(_GRID,),
        in_specs=[
            pl.BlockSpec((_BR, H), lambda i: (i, 0)),
            pl.BlockSpec((_BR, H), lambda i: (i, 0)),
            pl.BlockSpec((2, _BR, H), lambda i: (0, i, 0)),
            pl.BlockSpec((2, _BR, 1), lambda i: (0, i, 0)),
            pl.BlockSpec((_BR, 1), lambda i: (i, 0)),
            pl.BlockSpec((H, C), lambda i: (0, 0)),
            pl.BlockSpec((1, C), lambda i: (0, 0)),
        ],
        out_specs=pl.BlockSpec((_BR, C), lambda i: (i, 0)),
        out_shape=jax.ShapeDtypeStruct((N, C), _f32),
    )(layer_in, raw, part, svec, cntinv, W2, b2r)


def kernel(h, edge_index, W1, b1, Wg0, bg0, Wg1, bg1, W2, b2):
    rows = jnp.pad(edge_index[0], (0, EP - E), constant_values=N)
    cols = jnp.pad(edge_index[1], (0, EP - E), constant_values=N)
    rc = jnp.stack([rows.reshape(NW, NB, B), cols.reshape(NW, NB, B)],
                   axis=2).reshape(NW * NB, 2, B)

    degs, cnts = _hist(rc)
    dinv2, cntinv = _tc_prep(degs.reshape(2, NP, 1), cnts.reshape(2, NP, 1))
    dinv = dinv2.reshape(NP)

    Wg0cat = jnp.concatenate([Wg0[:H], Wg0[H:]], axis=1)
    Wg1cat = jnp.concatenate([Wg1[:H], Wg1[H:]], axis=1)
    bgv0 = jnp.stack([jnp.zeros((), _f32), bg0[0]]).reshape(1, 2)
    bgv1 = jnp.stack([jnp.zeros((), _f32), bg1[0]]).reshape(1, 2)

    hh, g0 = _tc_mm0(h, W1, b1.reshape(1, H), Wg0cat, bgv0)

    part1, s1 = _edge(rc, hh, g0[:, 0], g0[:, 1], dinv)

    hh2, g1 = _tc_comb(hh, hh, part1, s1.reshape(2, NP, 1),
                       cntinv, Wg1cat, bgv1)

    part2, s2 = _edge(rc, hh2, g1[:, 0], g1[:, 1], dinv)

    return _tc_final(hh2, hh, part2, s2.reshape(2, NP, 1),
                     cntinv, W2, b2.reshape(1, C))


# speedup vs baseline: 1.2208x; 1.2208x over previous
import functools

import jax
import jax.numpy as jnp
from jax import lax
from jax.experimental import pallas as pl
from jax.experimental.pallas import tpu as pltpu
from jax.experimental.pallas import tpu_sc as plsc

N = 10000
E = 320000
H = 128
C = 16
EPS = 0.5

NP = 10112
NW = 32
B = 128
NB0 = 54
NB1 = 104
BLKS = 16 * (NB0 + NB1)
EP = BLKS * B
NBH = BLKS // NW
RPT = NP // 16

_mesh = plsc.VectorSubcoreMesh(
    core_axis_name="c", subcore_axis_name="s", num_cores=2, num_subcores=16)

_f32 = jnp.float32


def _hist_body(rc_hbm, deg_out, cnt_out, dacc, cacc, idxb, ones_v):
    cid = lax.axis_index("c")
    sid = lax.axis_index("s")
    wid = cid * 16 + sid
    zero_row = jnp.zeros((16,), _f32)
    one_row = jnp.ones((16,), _f32)

    for g in range(B // 16):
        ones_v[pl.ds(g * 16, 16)] = zero_row
    r0 = sid * RPT
    for k in range(4):
        pltpu.sync_copy(ones_v.at[pl.ds(0, B)], dacc.at[pl.ds(r0 + k * B, B)])
        pltpu.sync_copy(ones_v.at[pl.ds(0, B)], cacc.at[pl.ds(r0 + k * B, B)])
    pltpu.sync_copy(ones_v.at[pl.ds(0, RPT - 4 * B)],
                    dacc.at[pl.ds(r0 + 4 * B, RPT - 4 * B)])
    pltpu.sync_copy(ones_v.at[pl.ds(0, RPT - 4 * B)],
                    cacc.at[pl.ds(r0 + 4 * B, RPT - 4 * B)])
    for g in range(B // 16):
        ones_v[pl.ds(g * 16, 16)] = one_row
    plsc.subcore_barrier()

    def _blk(k, carry):
        pltpu.sync_copy(rc_hbm.at[wid * NBH + k], idxb)
        pltpu.sync_copy(ones_v.at[pl.ds(0, B)], dacc.at[idxb.at[0]],
                        add=True)
        pltpu.sync_copy(ones_v.at[pl.ds(0, B)], cacc.at[idxb.at[1]],
                        add=True)
        return carry
    lax.fori_loop(0, NBH, _blk, 0)
    plsc.subcore_barrier()

    pltpu.sync_copy(dacc.at[pl.ds(r0, RPT)], deg_out.at[cid, pl.ds(r0, RPT)])
    pltpu.sync_copy(cacc.at[pl.ds(r0, RPT)], cnt_out.at[cid, pl.ds(r0, RPT)])


_hist = functools.partial(
    pl.kernel,
    out_type=(jax.ShapeDtypeStruct((2, NP), _f32),
              jax.ShapeDtypeStruct((2, NP), _f32)),
    mesh=_mesh,
    compiler_params=pltpu.CompilerParams(
        needs_layout_passes=False, use_tc_tiling_on_sc=False),
    scratch_types=[
        pltpu.MemorySpace.VMEM_SHARED((NP,), _f32),
        pltpu.MemorySpace.VMEM_SHARED((NP,), _f32),
        pltpu.VMEM((2, B), jnp.int32),
        pltpu.VMEM((B,), _f32),
    ],
)(_hist_body)


def _edge_body(rc_hbm, table_hbm, a_hbm, b_hbm, d_hbm,
               part_out, s_out, accp, accs, a_sp, b_sp, d_sp,
               idxb, gbuf, arb, bcb, drb, dcb, coef_v, st_v,
               gsem0, gsem1):
    cid = lax.axis_index("c")
    sid = lax.axis_index("s")
    wid = cid * 16 + sid
    zero_row = jnp.zeros((16,), _f32)
    r0 = sid * RPT

    ch_h = pl.ds(r0, RPT)
    for (hbm, sp) in ((a_hbm, a_sp), (b_hbm, b_sp), (d_hbm, d_sp)):
        pltpu.sync_copy(hbm.at[ch_h], st_v)
        pltpu.sync_copy(st_v, sp.at[ch_h])

    def _zrow(i, carry):
        for j in range(8):
            gbuf[0, i, pl.ds(j * 16, 16)] = zero_row
        return carry
    lax.fori_loop(0, B, _zrow, 0)
    for g in range(B // 16):
        coef_v[0, pl.ds(g * 16, 16)] = zero_row
    for k in range(4):
        pltpu.sync_copy(gbuf.at[0], accp.at[pl.ds(r0 + k * B, B)])
        pltpu.sync_copy(coef_v.at[0], accs.at[pl.ds(r0 + k * B, B)])
    pltpu.sync_copy(gbuf.at[0, pl.ds(0, RPT - 4 * B)],
                    accp.at[pl.ds(r0 + 4 * B, RPT - 4 * B)])
    pltpu.sync_copy(coef_v.at[0, pl.ds(0, RPT - 4 * B)],
                    accs.at[pl.ds(r0 + 4 * B, RPT - 4 * B)])
    plsc.subcore_barrier()

    def _tg_desc(slot, p, gsem):
        return pltpu.make_async_copy(table_hbm.at[idxb.at[slot, 0]],
                                     gbuf.at[p], gsem)

    lane = lax.broadcasted_iota(jnp.int32, (16,), 0)
    z16 = lane * 0
    o16 = z16 + 1

    nb = jnp.where(cid == 0, NB0, NB1)
    blk0 = jnp.where(cid == 0, sid * NB0, 16 * NB0 + sid * NB1)

    pltpu.sync_copy(rc_hbm.at[blk0], idxb.at[0])
    _tg_desc(0, 0, gsem0).start()

    def _kk(kk, carry):
        for ph in range(2):
            k = 2 * kk + ph
            p = ph
            q = 1 - ph
            gsem_p = gsem0 if p == 0 else gsem1
            gsem_q = gsem1 if p == 0 else gsem0

            kn = jnp.minimum(k + 1, nb - 1)
            pltpu.sync_copy(rc_hbm.at[blk0 + kn], idxb.at[q])
            _tg_desc(q, q, gsem_q).start()

            pltpu.sync_copy(a_sp.at[idxb.at[p, 0]], arb)
            pltpu.sync_copy(d_sp.at[idxb.at[p, 0]], drb)
            pltpu.sync_copy(b_sp.at[idxb.at[p, 1]], bcb)
            pltpu.sync_copy(d_sp.at[idxb.at[p, 1]], dcb)

            for g in range(B // 16):
                sl = pl.ds(g * 16, 16)
                xv = arb[sl] + bcb[sl]
                ex = jnp.exp(-2.0 * jnp.abs(xv))
                thv = jnp.sign(xv) * (1.0 - ex) / (1.0 + ex)
                coef_v[p, sl] = thv * (drb[sl] * dcb[sl])

            _tg_desc(p, p, gsem_p).wait()

            def _scaleg(g, carry2):
                cvec = coef_v[p, pl.ds(g * 16, 16)]
                for e16 in range(16):
                    e = g * 16 + e16
                    cf = cvec[e16]
                    for j in range(8):
                        gbuf[p, e, pl.ds(j * 16, 16)] = (
                            gbuf[p, e, pl.ds(j * 16, 16)] * cf)
                return carry2
            lax.fori_loop(0, B // 16, _scaleg, 0)

            pltpu.sync_copy(gbuf.at[p], accp.at[idxb.at[p, 1]], add=True)
            pltpu.sync_copy(coef_v.at[p], accs.at[idxb.at[p, 1]], add=True)
        return carry
    lax.fori_loop(0, nb // 2, _kk, 0)
    _tg_desc(0, 0, gsem0).wait()
    plsc.subcore_barrier()

    pltpu.sync_copy(accp.at[pl.ds(r0, RPT)], part_out.at[cid, pl.ds(r0, RPT)])
    pltpu.sync_copy(accs.at[pl.ds(r0, RPT)], s_out.at[cid, pl.ds(r0, RPT)])


_edge = functools.partial(
    pl.kernel,
    out_type=(jax.ShapeDtypeStruct((2, NP, H), _f32),
              jax.ShapeDtypeStruct((2, NP), _f32)),
    mesh=_mesh,
    compiler_params=pltpu.CompilerParams(
        needs_layout_passes=False, use_tc_tiling_on_sc=False),
    scratch_types=[
        pltpu.MemorySpace.VMEM_SHARED((NP, H), _f32),
        pltpu.MemorySpace.VMEM_SHARED((NP,), _f32),
        pltpu.MemorySpace.VMEM_SHARED((NP,), _f32),
        pltpu.MemorySpace.VMEM_SHARED((NP,), _f32),
        pltpu.MemorySpace.VMEM_SHARED((NP,), _f32),
        pltpu.VMEM((2, 2, B), jnp.int32),
        pltpu.VMEM((2, B, H), _f32),
        pltpu.VMEM((B,), _f32),
        pltpu.VMEM((B,), _f32),
        pltpu.VMEM((B,), _f32),
        pltpu.VMEM((B,), _f32),
        pltpu.VMEM((2, B), _f32),
        pltpu.VMEM((RPT,), _f32),
        pltpu.SemaphoreType.DMA,
        pltpu.SemaphoreType.DMA,
    ],
)(_edge_body)


_GRID = 10
_BR = N // _GRID


def _mm0_body(h_ref, W1_ref, b1_ref, Wg_ref, bgv_ref, hh_ref, g_ref):
    hh = jnp.maximum(
        jnp.dot(h_ref[...], W1_ref[...], preferred_element_type=_f32)
        + b1_ref[...], 0.0)
    hh_ref[...] = hh
    g_ref[...] = jnp.dot(hh, Wg_ref[...], preferred_element_type=_f32) + bgv_ref[...]


def _tc_mm0(h, W1, b1r, Wgcat, bgv):
    return pl.pallas_call(
        _mm0_body,
        grid=(_GRID,),
        in_specs=[
            pl.BlockSpec((_BR, H), lambda i: (i, 0)),
            pl.BlockSpec((H, H), lambda i: (0, 0)),
            pl.BlockSpec((1, H), lambda i: (0, 0)),
            pl.BlockSpec((H, 2), lambda i: (0, 0)),
            pl.BlockSpec((1, 2), lambda i: (0, 0)),
        ],
        out_specs=[
            pl.BlockSpec((_BR, H), lambda i: (i, 0)),
            pl.BlockSpec((_BR, 2), lambda i: (i, 0)),
        ],
        out_shape=[
            jax.ShapeDtypeStruct((NP, H), _f32),
            jax.ShapeDtypeStruct((NP, 2), _f32),
        ],
    )(h, W1, b1r, Wgcat, bgv)


def _prep_body(d_ref, c_ref, dinv_ref, cnti_ref):
    deg = d_ref[0] + d_ref[1]
    cnt = c_ref[0] + c_ref[1]
    rowid = lax.broadcasted_iota(jnp.int32, (NP, 1), 0)
    dinv_ref[...] = jnp.where(
        rowid < N, lax.rsqrt(jnp.maximum(deg, 1.0)), 0.0)
    cnti_ref[...] = 1.0 / jnp.maximum(cnt, 1.0)


def _tc_prep(degs, cnts):
    return pl.pallas_call(
        _prep_body,
        out_shape=[
            jax.ShapeDtypeStruct((NP, 1), _f32),
            jax.ShapeDtypeStruct((NP, 1), _f32),
        ],
    )(degs, cnts)


def _comb_body(lin_ref, raw_ref, p_ref, s_ref, ci_ref, Wg_ref, bgv_ref,
               hh2_ref, g_ref):
    P = p_ref[0] + p_ref[1]
    S = s_ref[0] + s_ref[1]
    out = (S * lin_ref[...] + P) * ci_ref[...]
    hh2 = EPS * raw_ref[...] + out
    hh2_ref[...] = hh2
    g_ref[...] = jnp.dot(hh2, Wg_ref[...], preferred_element_type=_f32) + bgv_ref[...]


def _tc_comb(layer_in, raw, part, svec, cntinv, Wgcat, bgv):
    return pl.pallas_call(
        _comb_body,
        grid=(_GRID,),
        in_specs=[
            pl.BlockSpec((_BR, H), lambda i: (i, 0)),
            pl.BlockSpec((_BR, H), lambda i: (i, 0)),
            pl.BlockSpec((2, _BR, H), lambda i: (0, i, 0)),
            pl.BlockSpec((2, _BR, 1), lambda i: (0, i, 0)),
            pl.BlockSpec((_BR, 1), lambda i: (i, 0)),
            pl.BlockSpec((H, 2), lambda i: (0, 0)),
            pl.BlockSpec((1, 2), lambda i: (0, 0)),
        ],
        out_specs=[
            pl.BlockSpec((_BR, H), lambda i: (i, 0)),
            pl.BlockSpec((_BR, 2), lambda i: (i, 0)),
        ],
        out_shape=[
            jax.ShapeDtypeStruct((NP, H), _f32),
            jax.ShapeDtypeStruct((NP, 2), _f32),
        ],
    )(layer_in, raw, part, svec, cntinv, Wgcat, bgv)


def _final_body(lin_ref, raw_ref, p_ref, s_ref, ci_ref, W2_ref, b2_ref,
                out_ref):
    P = p_ref[0] + p_ref[1]
    S = s_ref[0] + s_ref[1]
    out = (S * lin_ref[...] + P) * ci_ref[...]
    hh3 = EPS * raw_ref[...] + out
    logits = jnp.dot(hh3, W2_ref[...], preferred_element_type=_f32) + b2_ref[...]
    m = jnp.max(logits, axis=1, keepdims=True)
    lse = m + jnp.log(jnp.sum(jnp.exp(logits - m), axis=1, keepdims=True))
    out_ref[...] = logits - lse


def _tc_final(layer_in, raw, part, svec, cntinv, W2, b2r):
    return pl.pallas_call(
        _final_body,
        grid=(_GRID,),
        in_specs=[
            pl.BlockSpec((_BR, H), lambda i: (i, 0)),
            pl.BlockSpec((_BR, H), lambda i: (i, 0)),
            pl.BlockSpec((2, _BR, H), lambda i: (0, i, 0)),
            pl.BlockSpec((2, _BR, 1), lambda i: (0, i, 0)),
            pl.BlockSpec((_BR, 1), lambda i: (i, 0)),
            pl.BlockSpec((H, C), lambda i: (0, 0)),
            pl.BlockSpec((1, C), lambda i: (0, 0)),
        ],
        out_specs=pl.BlockSpec((_BR, C), lambda i: (i, 0)),
        out_shape=jax.ShapeDtypeStruct((N, C), _f32),
    )(layer_in, raw, part, svec, cntinv, W2, b2r)


def kernel(h, edge_index, W1, b1, Wg0, bg0, Wg1, bg1, W2, b2):
    rows = jnp.pad(edge_index[0], (0, EP - E), constant_values=N)
    cols = jnp.pad(edge_index[1], (0, EP - E), constant_values=N)
    rc = jnp.stack([rows.reshape(BLKS, B), cols.reshape(BLKS, B)], axis=1)

    degs, cnts = _hist(rc)
    dinv2, cntinv = _tc_prep(degs.reshape(2, NP, 1), cnts.reshape(2, NP, 1))
    dinv = dinv2.reshape(NP)

    Wg0cat = jnp.concatenate([Wg0[:H], Wg0[H:]], axis=1)
    Wg1cat = jnp.concatenate([Wg1[:H], Wg1[H:]], axis=1)
    bgv0 = jnp.stack([jnp.zeros((), _f32), bg0[0]]).reshape(1, 2)
    bgv1 = jnp.stack([jnp.zeros((), _f32), bg1[0]]).reshape(1, 2)

    hh, g0 = _tc_mm0(h, W1, b1.reshape(1, H), Wg0cat, bgv0)

    part1, s1 = _edge(rc, hh, g0[:, 0], g0[:, 1], dinv)

    hh2, g1 = _tc_comb(hh, hh, part1, s1.reshape(2, NP, 1),
                       cntinv, Wg1cat, bgv1)

    part2, s2 = _edge(rc, hh2, g1[:, 0], g1[:, 1], dinv)

    return _tc_final(hh2, hh, part2, s2.reshape(2, NP, 1),
                     cntinv, W2, b2.reshape(1, C))


# speedup vs baseline: 1.5169x; 1.2426x over previous
import functools

import jax
import jax.numpy as jnp
from jax import lax
from jax.experimental import pallas as pl
from jax.experimental.pallas import tpu as pltpu
from jax.experimental.pallas import tpu_sc as plsc

N = 10000
E = 320000
H = 128
C = 16
EPS = 0.5

NP = 10112
NW = 32
B = 128
NB0 = 96
NB1 = 62
BLKS = 16 * (NB0 + NB1)
EP = BLKS * B
NBH = BLKS // NW
RPT = NP // 16

_mesh = plsc.VectorSubcoreMesh(
    core_axis_name="c", subcore_axis_name="s", num_cores=2, num_subcores=16)

_f32 = jnp.float32


def _hist_body(rc_hbm, deg_out, cnt_out, dacc, cacc, idxb, ones_v):
    cid = lax.axis_index("c")
    sid = lax.axis_index("s")
    wid = cid * 16 + sid
    zero_row = jnp.zeros((16,), _f32)
    one_row = jnp.ones((16,), _f32)

    for g in range(B // 16):
        ones_v[pl.ds(g * 16, 16)] = zero_row
    r0 = sid * RPT
    for k in range(4):
        pltpu.sync_copy(ones_v.at[pl.ds(0, B)], dacc.at[pl.ds(r0 + k * B, B)])
        pltpu.sync_copy(ones_v.at[pl.ds(0, B)], cacc.at[pl.ds(r0 + k * B, B)])
    pltpu.sync_copy(ones_v.at[pl.ds(0, RPT - 4 * B)],
                    dacc.at[pl.ds(r0 + 4 * B, RPT - 4 * B)])
    pltpu.sync_copy(ones_v.at[pl.ds(0, RPT - 4 * B)],
                    cacc.at[pl.ds(r0 + 4 * B, RPT - 4 * B)])
    for g in range(B // 16):
        ones_v[pl.ds(g * 16, 16)] = one_row
    plsc.subcore_barrier()

    def _blk(k, carry):
        pltpu.sync_copy(rc_hbm.at[wid * NBH + k], idxb)
        pltpu.sync_copy(ones_v.at[pl.ds(0, B)], dacc.at[idxb.at[0]],
                        add=True)
        pltpu.sync_copy(ones_v.at[pl.ds(0, B)], cacc.at[idxb.at[1]],
                        add=True)
        return carry
    lax.fori_loop(0, NBH, _blk, 0)
    plsc.subcore_barrier()

    pltpu.sync_copy(dacc.at[pl.ds(r0, RPT)], deg_out.at[cid, pl.ds(r0, RPT)])
    pltpu.sync_copy(cacc.at[pl.ds(r0, RPT)], cnt_out.at[cid, pl.ds(r0, RPT)])


_hist = functools.partial(
    pl.kernel,
    out_type=(jax.ShapeDtypeStruct((2, NP), _f32),
              jax.ShapeDtypeStruct((2, NP), _f32)),
    mesh=_mesh,
    compiler_params=pltpu.CompilerParams(
        needs_layout_passes=False, use_tc_tiling_on_sc=False),
    scratch_types=[
        pltpu.MemorySpace.VMEM_SHARED((NP,), _f32),
        pltpu.MemorySpace.VMEM_SHARED((NP,), _f32),
        pltpu.VMEM((2, B), jnp.int32),
        pltpu.VMEM((B,), _f32),
    ],
)(_hist_body)


def _edge_body(rc_hbm, table_hbm, a_hbm, b_hbm, d_hbm,
               part_out, s_out, accp, accs, a_sp, b_sp, d_sp,
               idxb, gbuf, arb, bcb, drb, dcb, coef_v, st_v,
               gsem0, gsem1):
    cid = lax.axis_index("c")
    sid = lax.axis_index("s")
    wid = cid * 16 + sid
    zero_row = jnp.zeros((16,), _f32)
    r0 = sid * RPT

    ch_h = pl.ds(r0, RPT)
    for (hbm, sp) in ((a_hbm, a_sp), (b_hbm, b_sp), (d_hbm, d_sp)):
        pltpu.sync_copy(hbm.at[ch_h], st_v)
        pltpu.sync_copy(st_v, sp.at[ch_h])

    def _zrow(i, carry):
        for j in range(8):
            gbuf[0, i, pl.ds(j * 16, 16)] = zero_row
        return carry
    lax.fori_loop(0, B, _zrow, 0)
    for g in range(B // 16):
        coef_v[0, pl.ds(g * 16, 16)] = zero_row
    for k in range(4):
        pltpu.sync_copy(gbuf.at[0], accp.at[pl.ds(r0 + k * B, B)])
        pltpu.sync_copy(coef_v.at[0], accs.at[pl.ds(r0 + k * B, B)])
    pltpu.sync_copy(gbuf.at[0, pl.ds(0, RPT - 4 * B)],
                    accp.at[pl.ds(r0 + 4 * B, RPT - 4 * B)])
    pltpu.sync_copy(coef_v.at[0, pl.ds(0, RPT - 4 * B)],
                    accs.at[pl.ds(r0 + 4 * B, RPT - 4 * B)])
    plsc.subcore_barrier()

    def _tg_desc(slot, p, gsem):
        return pltpu.make_async_copy(table_hbm.at[idxb.at[slot, 0]],
                                     gbuf.at[p], gsem)

    lane = lax.broadcasted_iota(jnp.int32, (16,), 0)
    z16 = lane * 0
    o16 = z16 + 1

    nb = jnp.where(cid == 0, NB0, NB1)
    blk0 = jnp.where(cid == 0, sid * NB0, 16 * NB0 + sid * NB1)

    pltpu.sync_copy(rc_hbm.at[blk0], idxb.at[0])
    _tg_desc(0, 0, gsem0).start()

    def _kk(kk, carry):
        for ph in range(2):
            k = 2 * kk + ph
            p = ph
            q = 1 - ph
            gsem_p = gsem0 if p == 0 else gsem1
            gsem_q = gsem1 if p == 0 else gsem0

            kn = jnp.minimum(k + 1, nb - 1)
            pltpu.sync_copy(rc_hbm.at[blk0 + kn], idxb.at[q])
            _tg_desc(q, q, gsem_q).start()

            pltpu.sync_copy(a_sp.at[idxb.at[p, 0]], arb)
            pltpu.sync_copy(d_sp.at[idxb.at[p, 0]], drb)
            pltpu.sync_copy(b_sp.at[idxb.at[p, 1]], bcb)
            pltpu.sync_copy(d_sp.at[idxb.at[p, 1]], dcb)

            for g in range(B // 16):
                sl = pl.ds(g * 16, 16)
                xv = arb[sl] + bcb[sl]
                ex = jnp.exp(-2.0 * jnp.abs(xv))
                thv = jnp.sign(xv) * (1.0 - ex) / (1.0 + ex)
                coef_v[p, sl] = thv * (drb[sl] * dcb[sl])

            _tg_desc(p, p, gsem_p).wait()

            def _scaleg(g, carry2):
                cvec = coef_v[p, pl.ds(g * 16, 16)]
                for e16 in range(16):
                    e = g * 16 + e16
                    cf = cvec[e16]
                    for j in range(8):
                        gbuf[p, e, pl.ds(j * 16, 16)] = (
                            gbuf[p, e, pl.ds(j * 16, 16)] * cf)
                return carry2
            lax.fori_loop(0, B // 16, _scaleg, 0)

            pltpu.sync_copy(gbuf.at[p], accp.at[idxb.at[p, 1]], add=True)
            pltpu.sync_copy(coef_v.at[p], accs.at[idxb.at[p, 1]], add=True)
        return carry
    lax.fori_loop(0, nb // 2, _kk, 0)
    _tg_desc(0, 0, gsem0).wait()
    plsc.subcore_barrier()

    pltpu.sync_copy(accp.at[pl.ds(r0, RPT)], part_out.at[cid, pl.ds(r0, RPT)])
    pltpu.sync_copy(accs.at[pl.ds(r0, RPT)], s_out.at[cid, pl.ds(r0, RPT)])


_edge = functools.partial(
    pl.kernel,
    out_type=(jax.ShapeDtypeStruct((2, NP, H), _f32),
              jax.ShapeDtypeStruct((2, NP), _f32)),
    mesh=_mesh,
    compiler_params=pltpu.CompilerParams(
        needs_layout_passes=False, use_tc_tiling_on_sc=False),
    scratch_types=[
        pltpu.MemorySpace.VMEM_SHARED((NP, H), _f32),
        pltpu.MemorySpace.VMEM_SHARED((NP,), _f32),
        pltpu.MemorySpace.VMEM_SHARED((NP,), _f32),
        pltpu.MemorySpace.VMEM_SHARED((NP,), _f32),
        pltpu.MemorySpace.VMEM_SHARED((NP,), _f32),
        pltpu.VMEM((2, 2, B), jnp.int32),
        pltpu.VMEM((2, B, H), _f32),
        pltpu.VMEM((B,), _f32),
        pltpu.VMEM((B,), _f32),
        pltpu.VMEM((B,), _f32),
        pltpu.VMEM((B,), _f32),
        pltpu.VMEM((2, B), _f32),
        pltpu.VMEM((RPT,), _f32),
        pltpu.SemaphoreType.DMA,
        pltpu.SemaphoreType.DMA,
    ],
)(_edge_body)


_GRID = 10
_BR = N // _GRID


def _mm0_body(h_ref, W1_ref, b1_ref, Wg_ref, bgv_ref, hh_ref, g_ref):
    hh = jnp.maximum(
        jnp.dot(h_ref[...], W1_ref[...], preferred_element_type=_f32)
        + b1_ref[...], 0.0)
    hh_ref[...] = hh
    g_ref[...] = jnp.dot(hh, Wg_ref[...], preferred_element_type=_f32) + bgv_ref[...]


def _tc_mm0(h, W1, b1r, Wgcat, bgv):
    return pl.pallas_call(
        _mm0_body,
        grid=(_GRID,),
        in_specs=[
            pl.BlockSpec((_BR, H), lambda i: (i, 0)),
            pl.BlockSpec((H, H), lambda i: (0, 0)),
            pl.BlockSpec((1, H), lambda i: (0, 0)),
            pl.BlockSpec((H, 2), lambda i: (0, 0)),
            pl.BlockSpec((1, 2), lambda i: (0, 0)),
        ],
        out_specs=[
            pl.BlockSpec((_BR, H), lambda i: (i, 0)),
            pl.BlockSpec((_BR, 2), lambda i: (i, 0)),
        ],
        out_shape=[
            jax.ShapeDtypeStruct((NP, H), _f32),
            jax.ShapeDtypeStruct((NP, 2), _f32),
        ],
    )(h, W1, b1r, Wgcat, bgv)


def _prep_body(d_ref, c_ref, dinv_ref, cnti_ref):
    deg = d_ref[0] + d_ref[1]
    cnt = c_ref[0] + c_ref[1]
    rowid = lax.broadcasted_iota(jnp.int32, (NP, 1), 0)
    dinv_ref[...] = jnp.where(
        rowid < N, lax.rsqrt(jnp.maximum(deg, 1.0)), 0.0)
    cnti_ref[...] = 1.0 / jnp.maximum(cnt, 1.0)


def _tc_prep(degs, cnts):
    return pl.pallas_call(
        _prep_body,
        out_shape=[
            jax.ShapeDtypeStruct((NP, 1), _f32),
            jax.ShapeDtypeStruct((NP, 1), _f32),
        ],
    )(degs, cnts)


def _comb_body(lin_ref, raw_ref, p_ref, s_ref, ci_ref, Wg_ref, bgv_ref,
               hh2_ref, g_ref):
    P = p_ref[0] + p_ref[1]
    S = s_ref[0] + s_ref[1]
    out = (S * lin_ref[...] + P) * ci_ref[...]
    hh2 = EPS * raw_ref[...] + out
    hh2_ref[...] = hh2
    g_ref[...] = jnp.dot(hh2, Wg_ref[...], preferred_element_type=_f32) + bgv_ref[...]


def _tc_comb(layer_in, raw, part, svec, cntinv, Wgcat, bgv):
    return pl.pallas_call(
        _comb_body,
        grid=(_GRID,),
        in_specs=[
            pl.BlockSpec((_BR, H), lambda i: (i, 0)),
            pl.BlockSpec((_BR, H), lambda i: (i, 0)),
            pl.BlockSpec((2, _BR, H), lambda i: (0, i, 0)),
            pl.BlockSpec((2, _BR, 1), lambda i: (0, i, 0)),
            pl.BlockSpec((_BR, 1), lambda i: (i, 0)),
            pl.BlockSpec((H, 2), lambda i: (0, 0)),
            pl.BlockSpec((1, 2), lambda i: (0, 0)),
        ],
        out_specs=[
            pl.BlockSpec((_BR, H), lambda i: (i, 0)),
            pl.BlockSpec((_BR, 2), lambda i: (i, 0)),
        ],
        out_shape=[
            jax.ShapeDtypeStruct((NP, H), _f32),
            jax.ShapeDtypeStruct((NP, 2), _f32),
        ],
    )(layer_in, raw, part, svec, cntinv, Wgcat, bgv)


def _final_body(lin_ref, raw_ref, p_ref, s_ref, ci_ref, W2_ref, b2_ref,
                out_ref):
    P = p_ref[0] + p_ref[1]
    S = s_ref[0] + s_ref[1]
    out = (S * lin_ref[...] + P) * ci_ref[...]
    hh3 = EPS * raw_ref[...] + out
    logits = jnp.dot(hh3, W2_ref[...], preferred_element_type=_f32) + b2_ref[...]
    m = jnp.max(logits, axis=1, keepdims=True)
    lse = m + jnp.log(jnp.sum(jnp.exp(logits - m), axis=1, keepdims=True))
    out_ref[...] = logits - lse


def _tc_final(layer_in, raw, part, svec, cntinv, W2, b2r):
    return pl.pallas_call(
        _final_body,
        grid=(_GRID,),
        in_specs=[
            pl.BlockSpec((_BR, H), lambda i: (i, 0)),
            pl.BlockSpec((_BR, H), lambda i: (i, 0)),
            pl.BlockSpec((2, _BR, H), lambda i: (0, i, 0)),
            pl.BlockSpec((2, _BR, 1), lambda i: (0, i, 0)),
            pl.BlockSpec((_BR, 1), lambda i: (i, 0)),
            pl.BlockSpec((H, C), lambda i: (0, 0)),
            pl.BlockSpec((1, C), lambda i: (0, 0)),
        ],
        out_specs=pl.BlockSpec((_BR, C), lambda i: (i, 0)),
        out_shape=jax.ShapeDtypeStruct((N, C), _f32),
    )(layer_in, raw, part, svec, cntinv, W2, b2r)


def kernel(h, edge_index, W1, b1, Wg0, bg0, Wg1, bg1, W2, b2):
    rows = jnp.pad(edge_index[0], (0, EP - E), constant_values=N)
    cols = jnp.pad(edge_index[1], (0, EP - E), constant_values=N)
    rc = jnp.stack([rows.reshape(BLKS, B), cols.reshape(BLKS, B)], axis=1)

    degs, cnts = _hist(rc)
    dinv2, cntinv = _tc_prep(degs.reshape(2, NP, 1), cnts.reshape(2, NP, 1))
    dinv = dinv2.reshape(NP)

    Wg0cat = jnp.concatenate([Wg0[:H], Wg0[H:]], axis=1)
    Wg1cat = jnp.concatenate([Wg1[:H], Wg1[H:]], axis=1)
    bgv0 = jnp.stack([jnp.zeros((), _f32), bg0[0]]).reshape(1, 2)
    bgv1 = jnp.stack([jnp.zeros((), _f32), bg1[0]]).reshape(1, 2)

    hh, g0 = _tc_mm0(h, W1, b1.reshape(1, H), Wg0cat, bgv0)

    part1, s1 = _edge(rc, hh, g0[:, 0], g0[:, 1], dinv)

    hh2, g1 = _tc_comb(hh, hh, part1, s1.reshape(2, NP, 1),
                       cntinv, Wg1cat, bgv1)

    part2, s2 = _edge(rc, hh2, g1[:, 0], g1[:, 1], dinv)

    return _tc_final(hh2, hh, part2, s2.reshape(2, NP, 1),
                     cntinv, W2, b2.reshape(1, C))


# speedup vs baseline: 1.5183x; 1.0009x over previous
import functools

import jax
import jax.numpy as jnp
from jax import lax
from jax.experimental import pallas as pl
from jax.experimental.pallas import tpu as pltpu
from jax.experimental.pallas import tpu_sc as plsc

N = 10000
E = 320000
H = 128
C = 16
EPS = 0.5

NP = 10112
NW = 32
B = 128
NB0 = 96
NB1 = 62
BLKS = 16 * (NB0 + NB1)
EP = BLKS * B
NBH = BLKS // NW
RPT = NP // 16

_mesh = plsc.VectorSubcoreMesh(
    core_axis_name="c", subcore_axis_name="s", num_cores=2, num_subcores=16)

_f32 = jnp.float32


def _hist_body(rc_hbm, deg_out, cnt_out, dacc, cacc, idxb, ones_v):
    cid = lax.axis_index("c")
    sid = lax.axis_index("s")
    wid = cid * 16 + sid
    zero_row = jnp.zeros((16,), _f32)
    one_row = jnp.ones((16,), _f32)

    for g in range(B // 16):
        ones_v[pl.ds(g * 16, 16)] = zero_row
    r0 = sid * RPT
    for k in range(4):
        pltpu.sync_copy(ones_v.at[pl.ds(0, B)], dacc.at[pl.ds(r0 + k * B, B)])
        pltpu.sync_copy(ones_v.at[pl.ds(0, B)], cacc.at[pl.ds(r0 + k * B, B)])
    pltpu.sync_copy(ones_v.at[pl.ds(0, RPT - 4 * B)],
                    dacc.at[pl.ds(r0 + 4 * B, RPT - 4 * B)])
    pltpu.sync_copy(ones_v.at[pl.ds(0, RPT - 4 * B)],
                    cacc.at[pl.ds(r0 + 4 * B, RPT - 4 * B)])
    for g in range(B // 16):
        ones_v[pl.ds(g * 16, 16)] = one_row
    plsc.subcore_barrier()

    def _blk(k, carry):
        pltpu.sync_copy(rc_hbm.at[wid * NBH + k], idxb)
        pltpu.sync_copy(ones_v.at[pl.ds(0, B)], dacc.at[idxb.at[0]],
                        add=True)
        pltpu.sync_copy(ones_v.at[pl.ds(0, B)], cacc.at[idxb.at[1]],
                        add=True)
        return carry
    lax.fori_loop(0, NBH, _blk, 0)
    plsc.subcore_barrier()

    pltpu.sync_copy(dacc.at[pl.ds(r0, RPT)], deg_out.at[cid, pl.ds(r0, RPT)])
    pltpu.sync_copy(cacc.at[pl.ds(r0, RPT)], cnt_out.at[cid, pl.ds(r0, RPT)])


_hist = functools.partial(
    pl.kernel,
    out_type=(jax.ShapeDtypeStruct((2, NP), _f32),
              jax.ShapeDtypeStruct((2, NP), _f32)),
    mesh=_mesh,
    compiler_params=pltpu.CompilerParams(
        needs_layout_passes=False, use_tc_tiling_on_sc=False),
    scratch_types=[
        pltpu.MemorySpace.VMEM_SHARED((NP,), _f32),
        pltpu.MemorySpace.VMEM_SHARED((NP,), _f32),
        pltpu.VMEM((2, B), jnp.int32),
        pltpu.VMEM((B,), _f32),
    ],
)(_hist_body)


def _edge_body(rc_hbm, table_hbm, a_hbm, b_hbm, d_hbm,
               part_out, s_out, accp, accs, a_sp, b_sp, d_sp,
               idxb, gbuf, arb, bcb, drb, dcb, coef_v, st_v,
               gsem0, gsem1):
    cid = lax.axis_index("c")
    sid = lax.axis_index("s")
    wid = cid * 16 + sid
    zero_row = jnp.zeros((16,), _f32)
    r0 = sid * RPT

    ch_h = pl.ds(r0, RPT)
    for (hbm, sp) in ((a_hbm, a_sp), (b_hbm, b_sp), (d_hbm, d_sp)):
        pltpu.sync_copy(hbm.at[ch_h], st_v)
        pltpu.sync_copy(st_v, sp.at[ch_h])

    def _zrow(i, carry):
        for j in range(8):
            gbuf[0, i, pl.ds(j * 16, 16)] = zero_row
        return carry
    lax.fori_loop(0, B, _zrow, 0)
    for g in range(B // 16):
        coef_v[0, pl.ds(g * 16, 16)] = zero_row
    for k in range(4):
        pltpu.sync_copy(gbuf.at[0], accp.at[pl.ds(r0 + k * B, B)])
        pltpu.sync_copy(coef_v.at[0], accs.at[pl.ds(r0 + k * B, B)])
    pltpu.sync_copy(gbuf.at[0, pl.ds(0, RPT - 4 * B)],
                    accp.at[pl.ds(r0 + 4 * B, RPT - 4 * B)])
    pltpu.sync_copy(coef_v.at[0, pl.ds(0, RPT - 4 * B)],
                    accs.at[pl.ds(r0 + 4 * B, RPT - 4 * B)])
    plsc.subcore_barrier()

    def _tg_desc(slot, p, gsem):
        return pltpu.make_async_copy(table_hbm.at[idxb.at[slot, 0]],
                                     gbuf.at[p], gsem)

    nb = jnp.where(cid == 0, NB0, NB1)
    blk0 = jnp.where(cid == 0, sid * NB0, 16 * NB0 + sid * NB1)

    pltpu.sync_copy(rc_hbm.at[blk0], idxb.at[0])
    _tg_desc(0, 0, gsem0).start()

    def _kk(kk, carry):
        for ph in range(2):
            k = 2 * kk + ph
            p = ph
            q = 1 - ph
            gsem_p = gsem0 if p == 0 else gsem1
            gsem_q = gsem1 if p == 0 else gsem0

            kn = jnp.minimum(k + 1, nb - 1)
            pltpu.sync_copy(rc_hbm.at[blk0 + kn], idxb.at[q])
            _tg_desc(q, q, gsem_q).start()

            pltpu.sync_copy(a_sp.at[idxb.at[p, 0]], arb)
            pltpu.sync_copy(d_sp.at[idxb.at[p, 0]], drb)
            pltpu.sync_copy(b_sp.at[idxb.at[p, 1]], bcb)
            pltpu.sync_copy(d_sp.at[idxb.at[p, 1]], dcb)

            for g in range(B // 16):
                sl = pl.ds(g * 16, 16)
                xv = arb[sl] + bcb[sl]
                ex = jnp.exp(-2.0 * jnp.abs(xv))
                thv = jnp.sign(xv) * (1.0 - ex) / (1.0 + ex)
                coef_v[p, sl] = thv * (drb[sl] * dcb[sl])

            _tg_desc(p, p, gsem_p).wait()

            def _scaleg(g, carry2):
                cvec = coef_v[p, pl.ds(g * 16, 16)]
                for e16 in range(16):
                    e = g * 16 + e16
                    cf = cvec[e16]
                    for j in range(8):
                        gbuf[p, e, pl.ds(j * 16, 16)] = (
                            gbuf[p, e, pl.ds(j * 16, 16)] * cf)
                return carry2
            lax.fori_loop(0, B // 16, _scaleg, 0)

            pltpu.sync_copy(gbuf.at[p], accp.at[idxb.at[p, 1]], add=True)
            pltpu.sync_copy(coef_v.at[p], accs.at[idxb.at[p, 1]], add=True)
        return carry
    lax.fori_loop(0, nb // 2, _kk, 0)
    _tg_desc(0, 0, gsem0).wait()
    plsc.subcore_barrier()

    pltpu.sync_copy(accp.at[pl.ds(r0, RPT)], part_out.at[cid, pl.ds(r0, RPT)])
    pltpu.sync_copy(accs.at[pl.ds(r0, RPT)], s_out.at[cid, pl.ds(r0, RPT)])


_edge = functools.partial(
    pl.kernel,
    out_type=(jax.ShapeDtypeStruct((2, NP, H), _f32),
              jax.ShapeDtypeStruct((2, NP), _f32)),
    mesh=_mesh,
    compiler_params=pltpu.CompilerParams(
        needs_layout_passes=False, use_tc_tiling_on_sc=False),
    scratch_types=[
        pltpu.MemorySpace.VMEM_SHARED((NP, H), _f32),
        pltpu.MemorySpace.VMEM_SHARED((NP,), _f32),
        pltpu.MemorySpace.VMEM_SHARED((NP,), _f32),
        pltpu.MemorySpace.VMEM_SHARED((NP,), _f32),
        pltpu.MemorySpace.VMEM_SHARED((NP,), _f32),
        pltpu.VMEM((2, 2, B), jnp.int32),
        pltpu.VMEM((2, B, H), _f32),
        pltpu.VMEM((B,), _f32),
        pltpu.VMEM((B,), _f32),
        pltpu.VMEM((B,), _f32),
        pltpu.VMEM((B,), _f32),
        pltpu.VMEM((2, B), _f32),
        pltpu.VMEM((RPT,), _f32),
        pltpu.SemaphoreType.DMA,
        pltpu.SemaphoreType.DMA,
    ],
)(_edge_body)


_GRID = 10
_BR = N // _GRID


def _mm0_body(h_ref, W1_ref, b1_ref, Wg_ref, bgv_ref, hh_ref, g_ref):
    hh = jnp.maximum(
        jnp.dot(h_ref[...], W1_ref[...], preferred_element_type=_f32)
        + b1_ref[...], 0.0)
    hh_ref[...] = hh
    g_ref[...] = jnp.dot(hh, Wg_ref[...], preferred_element_type=_f32) + bgv_ref[...]


def _tc_mm0(h, W1, b1r, Wgcat, bgv):
    return pl.pallas_call(
        _mm0_body,
        grid=(_GRID,),
        in_specs=[
            pl.BlockSpec((_BR, H), lambda i: (i, 0)),
            pl.BlockSpec((H, H), lambda i: (0, 0)),
            pl.BlockSpec((1, H), lambda i: (0, 0)),
            pl.BlockSpec((H, 2), lambda i: (0, 0)),
            pl.BlockSpec((1, 2), lambda i: (0, 0)),
        ],
        out_specs=[
            pl.BlockSpec((_BR, H), lambda i: (i, 0)),
            pl.BlockSpec((_BR, 2), lambda i: (i, 0)),
        ],
        out_shape=[
            jax.ShapeDtypeStruct((NP, H), _f32),
            jax.ShapeDtypeStruct((NP, 2), _f32),
        ],
    )(h, W1, b1r, Wgcat, bgv)


def _prep_body(d_ref, c_ref, dinv_ref, cnti_ref):
    deg = d_ref[0] + d_ref[1]
    cnt = c_ref[0] + c_ref[1]
    rowid = lax.broadcasted_iota(jnp.int32, (NP, 1), 0)
    dinv_ref[...] = jnp.where(
        rowid < N, lax.rsqrt(jnp.maximum(deg, 1.0)), 0.0)
    cnti_ref[...] = 1.0 / jnp.maximum(cnt, 1.0)


def _tc_prep(degs, cnts):
    return pl.pallas_call(
        _prep_body,
        out_shape=[
            jax.ShapeDtypeStruct((NP, 1), _f32),
            jax.ShapeDtypeStruct((NP, 1), _f32),
        ],
    )(degs, cnts)


def _comb_body(lin_ref, raw_ref, p_ref, s_ref, ci_ref, Wg_ref, bgv_ref,
               hh2_ref, g_ref):
    P = p_ref[0] + p_ref[1]
    S = s_ref[0] + s_ref[1]
    out = (S * lin_ref[...] + P) * ci_ref[...]
    hh2 = EPS * raw_ref[...] + out
    hh2_ref[...] = hh2
    g_ref[...] = jnp.dot(hh2, Wg_ref[...], preferred_element_type=_f32) + bgv_ref[...]


def _tc_comb(layer_in, raw, part, svec, cntinv, Wgcat, bgv):
    return pl.pallas_call(
        _comb_body,
        grid=(_GRID,),
        in_specs=[
            pl.BlockSpec((_BR, H), lambda i: (i, 0)),
            pl.BlockSpec((_BR, H), lambda i: (i, 0)),
            pl.BlockSpec((2, _BR, H), lambda i: (0, i, 0)),
            pl.BlockSpec((2, _BR, 1), lambda i: (0, i, 0)),
            pl.BlockSpec((_BR, 1), lambda i: (i, 0)),
            pl.BlockSpec((H, 2), lambda i: (0, 0)),
            pl.BlockSpec((1, 2), lambda i: (0, 0)),
        ],
        out_specs=[
            pl.BlockSpec((_BR, H), lambda i: (i, 0)),
            pl.BlockSpec((_BR, 2), lambda i: (i, 0)),
        ],
        out_shape=[
            jax.ShapeDtypeStruct((NP, H), _f32),
            jax.ShapeDtypeStruct((NP, 2), _f32),
        ],
    )(layer_in, raw, part, svec, cntinv, Wgcat, bgv)


def _final_body(lin_ref, raw_ref, p_ref, s_ref, ci_ref, W2_ref, b2_ref,
                out_ref):
    P = p_ref[0] + p_ref[1]
    S = s_ref[0] + s_ref[1]
    out = (S * lin_ref[...] + P) * ci_ref[...]
    hh3 = EPS * raw_ref[...] + out
    logits = jnp.dot(hh3, W2_ref[...], preferred_element_type=_f32) + b2_ref[...]
    m = jnp.max(logits, axis=1, keepdims=True)
    lse = m + jnp.log(jnp.sum(jnp.exp(logits - m), axis=1, keepdims=True))
    out_ref[...] = logits - lse


def _tc_final(layer_in, raw, part, svec, cntinv, W2, b2r):
    return pl.pallas_call(
        _final_body,
        grid=(_GRID,),
        in_specs=[
            pl.BlockSpec((_BR, H), lambda i: (i, 0)),
            pl.BlockSpec((_BR, H), lambda i: (i, 0)),
            pl.BlockSpec((2, _BR, H), lambda i: (0, i, 0)),
            pl.BlockSpec((2, _BR, 1), lambda i: (0, i, 0)),
            pl.BlockSpec((_BR, 1), lambda i: (i, 0)),
            pl.BlockSpec((H, C), lambda i: (0, 0)),
            pl.BlockSpec((1, C), lambda i: (0, 0)),
        ],
        out_specs=pl.BlockSpec((_BR, C), lambda i: (i, 0)),
        out_shape=jax.ShapeDtypeStruct((N, C), _f32),
    )(layer_in, raw, part, svec, cntinv, W2, b2r)


def kernel(h, edge_index, W1, b1, Wg0, bg0, Wg1, bg1, W2, b2):
    rows = jnp.pad(edge_index[0], (0, EP - E), constant_values=N)
    cols = jnp.pad(edge_index[1], (0, EP - E), constant_values=N)
    rc = jnp.stack([rows.reshape(BLKS, B), cols.reshape(BLKS, B)], axis=1)

    degs, cnts = _hist(rc)
    dinv2, cntinv = _tc_prep(degs.reshape(2, NP, 1), cnts.reshape(2, NP, 1))
    dinv = dinv2.reshape(NP)

    Wg0cat = jnp.concatenate([Wg0[:H], Wg0[H:]], axis=1)
    Wg1cat = jnp.concatenate([Wg1[:H], Wg1[H:]], axis=1)
    bgv0 = jnp.stack([jnp.zeros((), _f32), bg0[0]]).reshape(1, 2)
    bgv1 = jnp.stack([jnp.zeros((), _f32), bg1[0]]).reshape(1, 2)

    hh, g0 = _tc_mm0(h, W1, b1.reshape(1, H), Wg0cat, bgv0)

    part1, s1 = _edge(rc, hh, g0[:, 0], g0[:, 1], dinv)

    hh2, g1 = _tc_comb(hh, hh, part1, s1.reshape(2, NP, 1),
                       cntinv, Wg1cat, bgv1)

    part2, s2 = _edge(rc, hh2, g1[:, 0], g1[:, 1], dinv)

    return _tc_final(hh2, hh, part2, s2.reshape(2, NP, 1),
                     cntinv, W2, b2.reshape(1, C))
